# baseline (speedup 1.0000x reference)
import functools

import jax
import jax.numpy as jnp
from jax import lax
from jax.experimental import pallas as pl
from jax.experimental.pallas import tpu as pltpu
from jax.experimental.pallas import tpu_sc as plsc


@functools.lru_cache(maxsize=None)
def _make_gather(V, D, B0, B1):
    info = plsc.get_sparse_core_info()
    NC, NS, L = info.num_cores, info.num_subcores, info.num_lanes
    NW = NC * NS
    assert B0 % NW == 0 and D == 64 and B1 % 2 == 0
    G = B0 // NW
    assert G == 128
    mesh = plsc.VectorSubcoreMesh(core_axis_name="c", subcore_axis_name="s")

    DT, DS = D // 8, 8

    @functools.partial(
        pl.kernel,
        mesh=mesh,
        out_type=jax.ShapeDtypeStruct((B1, DT, NW, DS, G), jnp.float32),
        scratch_types=[
            pltpu.VMEM((B1 * G,), jnp.int32),
            [pltpu.VMEM((G, D), jnp.float32) for _ in range(2)],
            [pltpu.VMEM((1, DT, 1, DS, G), jnp.float32) for _ in range(2)],
            [pltpu.SemaphoreType.DMA for _ in range(2)],
            [pltpu.SemaphoreType.DMA for _ in range(2)],
        ],
        compiler_params=pltpu.CompilerParams(
            use_tc_tiling_on_sc=True, needs_layout_passes=False
        ),
    )
    def k(table_hbm, idx_hbm, out_hbm, idx_v, stag, blk, sem_g, sem_s):
        wid = lax.axis_index("s") * NC + lax.axis_index("c")
        base = wid * (B1 * G)
        pltpu.sync_copy(idx_hbm.at[pl.ds(base, B1 * G)], idx_v)
        lanes = lax.iota(jnp.int32, L)
        zeros = jnp.zeros((L,), jnp.int32)

        def start_gather(b1, b):
            def grp(g, _):
                pos16 = (g * L + lanes) * B1 + b1
                i16 = plsc.load_gather(idx_v, [pos16])
                for l in range(L):
                    pltpu.async_copy(
                        table_hbm.at[pl.ds(i16[l], 1)],
                        stag[b].at[pl.ds(g * L + l, 1)],
                        sem_g[b],
                    )
                return 0

            lax.fori_loop(0, G // L, grp, 0)

        def drain_gather(b):
            def w(p, _):
                pltpu.make_async_copy(
                    table_hbm.at[pl.ds(0, 1)], stag[b].at[pl.ds(0, 1)], sem_g[b]
                ).wait()
                return 0

            lax.fori_loop(0, G, w, 0)

        def permute(b):
            def row(q, _):
                for v in range(D // L):
                    sv = stag[b][q, pl.ds(v * L, L)]
                    d = v * L + lanes
                    plsc.store_scatter(
                        blk[b], [zeros, d // DS, zeros, d % DS, zeros + q], sv
                    )
                return 0

            lax.fori_loop(0, G, row, 0)

        def start_store(b1, b):
            return pltpu.async_copy(
                blk[b],
                out_hbm.at[pl.ds(b1, 1), :, pl.ds(wid, 1), :, :],
                sem_s[b],
            )

        def wait_store(b):
            pltpu.make_async_copy(
                blk[b],
                out_hbm.at[pl.ds(0, 1), :, pl.ds(0, 1), :, :],
                sem_s[b],
            ).wait()

        start_gather(0, 0)
        start_gather(1, 1)
        for b in range(2):
            drain_gather(b)
            permute(b)
            start_store(b, b)
            start_gather(b + 2, b)

        def steady(j, _):
            for b in range(2):
                b1 = 2 * j + b
                drain_gather(b)
                wait_store(b)
                permute(b)
                start_store(b1, b)

                @pl.when(b1 + 2 < B1)
                def _():
                    start_gather(b1 + 2, b)

            return 0

        lax.fori_loop(1, B1 // 2, steady, 0)
        for b in range(2):
            wait_store(b)

    return k


def kernel(input_, weight):
    B0, B1 = input_.shape
    V, D = weight.shape
    idx = input_.reshape(B0 * B1).astype(jnp.int32)
    x2 = _make_gather(V, D, B0, B1)(weight, idx)
    return x2.transpose(2, 4, 0, 1, 3).reshape(B0, B1, D)
